# baseline (speedup 1.0000x reference)
import jax
import jax.numpy as jnp
from jax.experimental import pallas as pl
from jax.experimental.pallas import tpu as pltpu

D_MODEL = 768
D_FF = 3072
N_EXPERTS = 8
TOP_K = 2

T_TILE = 256
F_TILE = 1536


def _moe_body(x_ref, gate_ref, w1_ref, w2_ref, y_ref, loss_ref,
              wt_ref, cnt_ref, psum_ref):
    t = pl.program_id(0)
    e = pl.program_id(1)
    f = pl.program_id(2)
    n_t = pl.num_programs(0)
    n_e = pl.num_programs(1)
    n_f = pl.num_programs(2)

    x = x_ref[...]

    @pl.when(jnp.logical_and(e == 0, f == 0))
    def _router():
        logits = jax.lax.dot_general(
            x, gate_ref[...], (((1,), (1,)), ((), ())),
            preferred_element_type=jnp.float32)
        m = jnp.max(logits, axis=1, keepdims=True)
        ex = jnp.exp(logits - m)
        probs = ex / jnp.sum(ex, axis=1, keepdims=True)
        iota = jax.lax.broadcasted_iota(jnp.int32, probs.shape, 1)
        big = jnp.int32(N_EXPERTS)
        m1 = jnp.max(probs, axis=1, keepdims=True)
        i1 = jnp.min(jnp.where(probs == m1, iota, big), axis=1, keepdims=True)
        sel1 = iota == i1
        p2 = jnp.where(sel1, jnp.float32(-1.0), probs)
        m2 = jnp.max(p2, axis=1, keepdims=True)
        i2 = jnp.min(jnp.where(p2 == m2, iota, big), axis=1, keepdims=True)
        sel2 = iota == i2
        wt = m1 * sel1.astype(jnp.float32) + m2 * sel2.astype(jnp.float32)
        wt_ref[...] = wt

        @pl.when(t == 0)
        def _init():
            cnt_ref[...] = jnp.zeros_like(cnt_ref)
            psum_ref[...] = jnp.zeros_like(psum_ref)

        onehots = sel1.astype(jnp.float32) + sel2.astype(jnp.float32)
        cnt_ref[...] += jnp.sum(onehots, axis=0, keepdims=True)
        psum_ref[...] += jnp.sum(probs, axis=0, keepdims=True)

    wt_all = wt_ref[...]
    col = jax.lax.broadcasted_iota(jnp.int32, wt_all.shape, 1) == e
    weight = jnp.sum(jnp.where(col, wt_all, 0.0), axis=1, keepdims=True)
    h = jax.lax.dot_general(
        x, w1_ref[0], (((1,), (1,)), ((), ())),
        preferred_element_type=jnp.float32)
    h = h * jax.nn.sigmoid(h)
    contrib = jax.lax.dot_general(
        weight * h, w2_ref[0], (((1,), (1,)), ((), ())),
        preferred_element_type=jnp.float32)

    @pl.when(jnp.logical_and(e == 0, f == 0))
    def _first():
        y_ref[...] = contrib

    @pl.when(jnp.logical_not(jnp.logical_and(e == 0, f == 0)))
    def _acc():
        y_ref[...] += contrib

    @pl.when(jnp.logical_and(t == n_t - 1,
                             jnp.logical_and(e == n_e - 1, f == n_f - 1)))
    def _loss():
        n_tok = n_t * T_TILE
        f_i = cnt_ref[...] / jnp.float32(n_tok * TOP_K)
        p_i = psum_ref[...] / jnp.float32(n_tok)
        loss_ref[...] = jnp.sum(f_i * p_i, keepdims=True).reshape(1, 1)


def kernel(x, gate_w, w1, w2):
    B, T, D = x.shape
    x2 = x.reshape(T, D)
    n_t = T // T_TILE
    n_f = D_FF // F_TILE

    grid = (n_t, N_EXPERTS, n_f)
    y, loss = pl.pallas_call(
        _moe_body,
        grid=grid,
        in_specs=[
            pl.BlockSpec((T_TILE, D), lambda t, e, f: (t, 0)),
            pl.BlockSpec((N_EXPERTS, D), lambda t, e, f: (0, 0)),
            pl.BlockSpec((1, F_TILE, D), lambda t, e, f: (e, f, 0)),
            pl.BlockSpec((1, D, F_TILE), lambda t, e, f: (e, 0, f)),
        ],
        out_specs=[
            pl.BlockSpec((T_TILE, D), lambda t, e, f: (t, 0)),
            pl.BlockSpec((1, 1), lambda t, e, f: (0, 0)),
        ],
        out_shape=[
            jax.ShapeDtypeStruct((T, D), jnp.float32),
            jax.ShapeDtypeStruct((1, 1), jnp.float32),
        ],
        scratch_shapes=[
            pltpu.VMEM((T_TILE, N_EXPERTS), jnp.float32),
            pltpu.VMEM((1, N_EXPERTS), jnp.float32),
            pltpu.VMEM((1, N_EXPERTS), jnp.float32),
        ],
    )(x2, gate_w, w1, w2)
    return y.reshape(B, T, D), loss[0, 0]


# speedup vs baseline: 1.1742x; 1.1742x over previous
import functools

import jax
import jax.numpy as jnp
from jax import lax
from jax.experimental import pallas as pl
from jax.experimental.pallas import tpu as pltpu
from jax.experimental.pallas import tpu_sc as plsc

D_MODEL = 768
D_FF = 3072
NE = 8
TOPK = 2
NT = 2048
NP = NT * TOPK
TILE = 128
PAD = NP + NE * TILE
NB = PAD // TILE
NGID = 48

NC, NS, L = 2, 16, 16
NW = NC * NS
TOK_W = NT // NW
SLOT_W = PAD // NW
GCHUNK = 40
VPP = NP // L


def _router_body(x_ref, g_ref, i1_ref, i2_ref, wa_ref, wb_ref, loss_ref):
    x = x_ref[...]
    logits = lax.dot_general(x, g_ref[...], (((1,), (1,)), ((), ())),
                             preferred_element_type=jnp.float32)
    m = jnp.max(logits, axis=1, keepdims=True)
    ex = jnp.exp(logits - m)
    probs = ex / jnp.sum(ex, axis=1, keepdims=True)
    iota = lax.broadcasted_iota(jnp.int32, probs.shape, 1)
    big = jnp.int32(NE)
    m1 = jnp.max(probs, axis=1, keepdims=True)
    i1 = jnp.min(jnp.where(probs == m1, iota, big), axis=1, keepdims=True)
    sel1 = iota == i1
    p2 = jnp.where(sel1, jnp.float32(-1.0), probs)
    m2 = jnp.max(p2, axis=1, keepdims=True)
    i2 = jnp.min(jnp.where(p2 == m2, iota, big), axis=1, keepdims=True)
    sel2 = iota == i2
    i1_ref[...] = i1
    i2_ref[...] = i2
    wa_ref[...] = m1
    wb_ref[...] = m2
    cnt = jnp.sum(sel1.astype(jnp.float32) + sel2.astype(jnp.float32),
                  axis=0, keepdims=True)
    psum = jnp.sum(probs, axis=0, keepdims=True)
    f_i = cnt / jnp.float32(NT * TOPK)
    p_i = psum / jnp.float32(NT)
    loss_ref[...] = jnp.sum(f_i * p_i, keepdims=True).reshape(1, 1)


def _router(x2, gate_w):
    return pl.pallas_call(
        _router_body,
        in_specs=[
            pl.BlockSpec((NT, D_MODEL), lambda: (0, 0)),
            pl.BlockSpec((NE, D_MODEL), lambda: (0, 0)),
        ],
        out_specs=[
            pl.BlockSpec((NT, 1), lambda: (0, 0)),
            pl.BlockSpec((NT, 1), lambda: (0, 0)),
            pl.BlockSpec((NT, 1), lambda: (0, 0)),
            pl.BlockSpec((NT, 1), lambda: (0, 0)),
            pl.BlockSpec((1, 1), lambda: (0, 0)),
        ],
        out_shape=[
            jax.ShapeDtypeStruct((NT, 1), jnp.int32),
            jax.ShapeDtypeStruct((NT, 1), jnp.int32),
            jax.ShapeDtypeStruct((NT, 1), jnp.float32),
            jax.ShapeDtypeStruct((NT, 1), jnp.float32),
            jax.ShapeDtypeStruct((1, 1), jnp.float32),
        ],
    )(x2, gate_w)


def _dispatch_body(i1_hbm, i2_hbm, x_hbm, xs_hbm, inva_hbm, invb_hbm,
                   gid_hbm, eids_v, rank_v, cnt_v, s_v, rids_v, gid_v,
                   rows_v, sem):
    wid = lax.axis_index("s") * NC + lax.axis_index("c")
    lanes = lax.broadcasted_iota(jnp.int32, (L,), 0)
    ones = jnp.ones((L,), jnp.int32)
    zeros = jnp.zeros((L,), jnp.int32)

    pltpu.sync_copy(i1_hbm, eids_v.at[pl.ds(0, NT)])
    pltpu.sync_copy(i2_hbm, eids_v.at[pl.ds(NT, NT)])
    cnt_v[...] = zeros

    def rank_step(i, _):
        off = pl.multiple_of(i * L, L)
        eid = eids_v[pl.ds(off, L)]
        base = plsc.load_gather(cnt_v, [eid])
        rin = zeros
        for e in range(NE):
            msk = eid == e
            c = plsc.cumsum(jnp.where(msk, 1, 0))
            rin = jnp.where(msk, c - 1, rin)
        rank_v[pl.ds(off, L)] = base + rin
        plsc.addupdate_scatter(cnt_v, [eid], ones)
        return 0

    lax.fori_loop(0, VPP, rank_step, 0)

    totals = cnt_v[...]
    padded = jnp.where(lanes < NE,
                       ((totals + (TILE - 1)) // TILE) * TILE, 0)
    prefix = plsc.cumsum(padded)
    s_v[...] = prefix
    shifted = plsc.load_gather(s_v, [jnp.maximum(lanes - 1, 0)])
    starts = jnp.where(lanes == 0, 0, shifted)
    s_v[...] = starts

    @pl.when(wid == 0)
    def _gids():
        for c in range(NGID // L):
            b = lanes + c * L
            acc = zeros
            for j in range(1, NE):
                sj = plsc.load_gather(s_v, [jnp.full((L,), j, jnp.int32)])
                acc = acc + jnp.where(b * TILE >= sj, 1, 0)
            gid_v[pl.ds(c * L, L)] = acc
        pltpu.sync_copy(gid_v, gid_hbm)

    def zero_step(i, _):
        rids_v[pl.ds(pl.multiple_of(i * L, L), L)] = zeros
        return 0

    lax.fori_loop(0, PAD // L, zero_step, 0)

    def dest_step(i, _):
        off = pl.multiple_of(i * L, L)
        eid = eids_v[pl.ds(off, L)]
        rank = rank_v[pl.ds(off, L)]
        sbase = plsc.load_gather(s_v, [eid])
        dest = sbase + rank
        rank_v[pl.ds(off, L)] = dest
        tok = (lanes + off) & (NT - 1)
        plsc.store_scatter(rids_v, [dest], tok)
        return 0

    lax.fori_loop(0, VPP, dest_step, 0)

    pltpu.sync_copy(rank_v.at[pl.ds(wid * TOK_W, TOK_W)],
                    inva_hbm.at[pl.ds(wid * TOK_W, TOK_W)])
    pltpu.sync_copy(rank_v.at[pl.ds(NT + wid * TOK_W, TOK_W)],
                    invb_hbm.at[pl.ds(wid * TOK_W, TOK_W)])

    for c in range(SLOT_W // GCHUNK):
        base = wid * SLOT_W + c * GCHUNK
        idx = rids_v.at[pl.ds(base, GCHUNK)]
        pltpu.async_copy(x_hbm.at[idx], rows_v, sem).wait()
        pltpu.sync_copy(rows_v, xs_hbm.at[pl.ds(base, GCHUNK)])


def _dispatch(i1, i2, x2):
    mesh = plsc.VectorSubcoreMesh(core_axis_name="c", subcore_axis_name="s")
    f = pl.kernel(
        _dispatch_body,
        out_type=[
            jax.ShapeDtypeStruct((PAD, D_MODEL), jnp.float32),
            jax.ShapeDtypeStruct((NT,), jnp.int32),
            jax.ShapeDtypeStruct((NT,), jnp.int32),
            jax.ShapeDtypeStruct((NGID,), jnp.int32),
        ],
        mesh=mesh,
        compiler_params=pltpu.CompilerParams(needs_layout_passes=False),
        scratch_types=[
            pltpu.VMEM((NP,), jnp.int32),
            pltpu.VMEM((NP,), jnp.int32),
            pltpu.VMEM((L,), jnp.int32),
            pltpu.VMEM((L,), jnp.int32),
            pltpu.VMEM((PAD,), jnp.int32),
            pltpu.VMEM((NGID,), jnp.int32),
            pltpu.VMEM((GCHUNK, D_MODEL), jnp.float32),
            pltpu.SemaphoreType.DMA,
        ],
    )
    return f(i1, i2, x2)


def _ffn_body(gid_ref, xs_ref, w1_ref, w2_ref, y_ref):
    xb = xs_ref[...].astype(jnp.bfloat16)
    h = lax.dot_general(xb, w1_ref[0], (((1,), (1,)), ((), ())),
                        preferred_element_type=jnp.float32)
    h = h * jax.nn.sigmoid(h)
    y_ref[...] = lax.dot_general(h.astype(jnp.bfloat16), w2_ref[0],
                                 (((1,), (1,)), ((), ())),
                                 preferred_element_type=jnp.float32)


def _ffn(gids, xs, w1b, w2b):
    grid_spec = pltpu.PrefetchScalarGridSpec(
        num_scalar_prefetch=1,
        grid=(NB,),
        in_specs=[
            pl.BlockSpec((TILE, D_MODEL), lambda b, g: (b, 0)),
            pl.BlockSpec((1, D_FF, D_MODEL), lambda b, g: (g[b], 0, 0)),
            pl.BlockSpec((1, D_MODEL, D_FF), lambda b, g: (g[b], 0, 0)),
        ],
        out_specs=pl.BlockSpec((TILE, D_MODEL), lambda b, g: (b, 0)),
    )
    return pl.pallas_call(
        _ffn_body,
        grid_spec=grid_spec,
        out_shape=jax.ShapeDtypeStruct((PAD, D_MODEL), jnp.float32),
    )(gids, xs, w1b, w2b)


def _combine_body(y_hbm, ia_hbm, ib_hbm, wa_hbm, wb_hbm, out_hbm,
                  ia_v, ib_v, wa_v, wb_v, ya_v, yb_v, o_v, sem):
    wid = lax.axis_index("s") * NC + lax.axis_index("c")
    half = TOK_W // 2
    for c in range(2):
        t0 = wid * TOK_W + c * half
        pltpu.sync_copy(ia_hbm.at[pl.ds(t0, half)], ia_v)
        pltpu.sync_copy(ib_hbm.at[pl.ds(t0, half)], ib_v)
        pltpu.sync_copy(wa_hbm.at[pl.ds(t0, half)], wa_v)
        pltpu.sync_copy(wb_hbm.at[pl.ds(t0, half)], wb_v)
        ca = pltpu.async_copy(y_hbm.at[ia_v], ya_v, sem)
        cb = pltpu.async_copy(y_hbm.at[ib_v], yb_v, sem)
        ca.wait()
        cb.wait()

        def row_step(r, _):
            sa = plsc.load_gather(wa_v, [jnp.full((L,), r, jnp.int32)])
            sb = plsc.load_gather(wb_v, [jnp.full((L,), r, jnp.int32)])
            for k in range(D_MODEL // L):
                o_v[r, pl.ds(k * L, L)] = (
                    ya_v[r, pl.ds(k * L, L)] * sa
                    + yb_v[r, pl.ds(k * L, L)] * sb)
            return 0

        lax.fori_loop(0, half, row_step, 0)
        pltpu.sync_copy(o_v, out_hbm.at[pl.ds(t0, half)])


def _combine(y, inva, invb, wa, wb):
    mesh = plsc.VectorSubcoreMesh(core_axis_name="c", subcore_axis_name="s")
    half = TOK_W // 2
    f = pl.kernel(
        _combine_body,
        out_type=jax.ShapeDtypeStruct((NT, D_MODEL), jnp.float32),
        mesh=mesh,
        compiler_params=pltpu.CompilerParams(needs_layout_passes=False),
        scratch_types=[
            pltpu.VMEM((half,), jnp.int32),
            pltpu.VMEM((half,), jnp.int32),
            pltpu.VMEM((half,), jnp.float32),
            pltpu.VMEM((half,), jnp.float32),
            pltpu.VMEM((half, D_MODEL), jnp.float32),
            pltpu.VMEM((half, D_MODEL), jnp.float32),
            pltpu.VMEM((half, D_MODEL), jnp.float32),
            pltpu.SemaphoreType.DMA,
        ],
    )
    return f(y, inva, invb, wa, wb)


def kernel(x, gate_w, w1, w2):
    B, T, D = x.shape
    x2 = x.reshape(B * T, D)
    i1, i2, wa, wb, loss = _router(x2, gate_w)
    xs, inva, invb, gids = _dispatch(i1.reshape(-1), i2.reshape(-1), x2)
    w1b = w1.astype(jnp.bfloat16)
    w2b = w2.astype(jnp.bfloat16)
    y = _ffn(gids[:NB], xs, w1b, w2b)
    out = _combine(y, inva, invb, wa.reshape(-1), wb.reshape(-1))
    return out.reshape(B, T, D), loss[0, 0]
